# baseline (speedup 1.0000x reference)
import jax
import jax.numpy as jnp
from jax.experimental import pallas as pl
from jax.experimental.pallas import tpu as pltpu

_N = 524288


def _put_kernel(x_hbm, o_hbm, vbuf, sem_big, sem_in, sem_out):
    big = pltpu.make_async_copy(x_hbm, o_hbm, sem_big)
    big.start()
    head_in = pltpu.make_async_copy(x_hbm.at[:, pl.ds(0, 128)], vbuf, sem_in)
    head_in.start()
    head_in.wait()
    lane = jax.lax.broadcasted_iota(jnp.int32, (48, 128), 1)
    row = jax.lax.broadcasted_iota(jnp.int32, (48, 128), 0)
    vbuf[...] = jnp.where(lane < 7, row.astype(jnp.float32), vbuf[...])
    big.wait()
    head_out = pltpu.make_async_copy(vbuf, o_hbm.at[:, pl.ds(0, 128)], sem_out)
    head_out.start()
    head_out.wait()


def kernel(x):
    xt = jnp.transpose(x, (1, 2, 0)).reshape(48, _N)
    y = pl.pallas_call(
        _put_kernel,
        in_specs=[pl.BlockSpec(memory_space=pl.ANY)],
        out_specs=pl.BlockSpec(memory_space=pl.ANY),
        out_shape=jax.ShapeDtypeStruct((48, _N), jnp.float32),
        scratch_shapes=[
            pltpu.VMEM((48, 128), jnp.float32),
            pltpu.SemaphoreType.DMA,
            pltpu.SemaphoreType.DMA,
            pltpu.SemaphoreType.DMA,
        ],
    )(xt)
    return jnp.transpose(y.reshape(6, 8, _N), (2, 0, 1))


# speedup vs baseline: 49.0987x; 49.0987x over previous
import jax
import jax.numpy as jnp
from jax.experimental import pallas as pl

_N = 524288
_RBLK = 8
_CBLK = _N // 2


def _put_kernel(x_ref, o_ref):
    ri = pl.program_id(0)
    ci = pl.program_id(1)
    o_ref[...] = x_ref[...]

    @pl.when(ci == 0)
    def _():
        lane = jax.lax.broadcasted_iota(jnp.int32, (_RBLK, 128), 1)
        row = jax.lax.broadcasted_iota(jnp.int32, (_RBLK, 128), 0) + ri * _RBLK
        o_ref[:, 0:128] = jnp.where(lane < 7, row.astype(jnp.float32),
                                    x_ref[:, 0:128])


def kernel(x):
    xt = jnp.transpose(x, (1, 2, 0)).reshape(48, _N)
    y = pl.pallas_call(
        _put_kernel,
        grid=(48 // _RBLK, _N // _CBLK),
        in_specs=[pl.BlockSpec((_RBLK, _CBLK), lambda r, c: (r, c))],
        out_specs=pl.BlockSpec((_RBLK, _CBLK), lambda r, c: (r, c)),
        out_shape=jax.ShapeDtypeStruct((48, _N), jnp.float32),
    )(xt)
    return jnp.transpose(y.reshape(6, 8, _N), (2, 0, 1))
